# baseline (speedup 1.0000x reference)
import functools

import jax
import jax.numpy as jnp
from jax import lax
from jax.experimental import pallas as pl
from jax.experimental.pallas import tpu as pltpu
from jax.experimental.pallas import tpu_sc as plsc

NN = 10000
EE = 320000
FD = 128
HD = 256
NC, NS = 2, 16
NW = NC * NS
EPW = EE // NW
EPWP = 10240
CH = 128
NCH = EPWP // CH
IR = EPWP // 128
ANN = 10240
RPS = ANN // NS
DUMP = 10200

_MESH = plsc.VectorSubcoreMesh(
    core_axis_name="c", subcore_axis_name="s", num_cores=NC, num_subcores=NS
)


def _fill(ref, nrows, ncols, value):
    v = jnp.full((16,), value, jnp.float32)

    def row(r, carry):
        for cc in range(ncols // 16):
            ref[r, pl.ds(cc * 16, 16)] = v
        return carry

    lax.fori_loop(0, nrows, row, 0)


@functools.partial(
    pl.kernel,
    out_type=jax.ShapeDtypeStruct((NC, ANN, FD), jnp.float32),
    mesh=_MESH,
    scratch_types=[
        pltpu.VMEM((IR, 128), jnp.int32),
        pltpu.VMEM((CH, FD), jnp.float32),
        pltpu.VMEM_SHARED((ANN, FD), jnp.float32),
    ],
)
def _deg_kernel(dsts_hbm, out_hbm, dst_v, ones_v, acc):
    c = lax.axis_index("c")
    s = lax.axis_index("s")
    w = c * NS + s
    _fill(ones_v, CH, FD, 0.0)
    for k in range(RPS // CH):
        pltpu.sync_copy(ones_v, acc.at[pl.ds(s * RPS + k * CH, CH)])
    _fill(ones_v, CH, FD, 1.0)
    pltpu.sync_copy(dsts_hbm.at[w], dst_v)
    plsc.subcore_barrier()

    def body(r, carry):
        pltpu.sync_copy(ones_v, acc.at[dst_v.at[r]], add=True)
        return carry

    lax.fori_loop(0, IR, body, 0)
    plsc.subcore_barrier()
    pltpu.sync_copy(acc.at[pl.ds(s * RPS, RPS)],
                    out_hbm.at[c].at[pl.ds(s * RPS, RPS)])


@functools.partial(
    pl.kernel,
    out_type=jax.ShapeDtypeStruct((NC, ANN, FD), jnp.float32),
    mesh=_MESH,
    scratch_types=[
        pltpu.VMEM((IR, 128), jnp.int32),
        pltpu.VMEM((IR, 128), jnp.int32),
        pltpu.VMEM((CH, FD), jnp.float32),
        pltpu.VMEM_SHARED((ANN, FD), jnp.float32),
        pltpu.SemaphoreType.DMA,
    ],
)
def _agg_kernel(xs_hbm, srcs_hbm, dsts_hbm, out_hbm,
                src_v, dst_v, rows0, acc, sg0):
    c = lax.axis_index("c")
    s = lax.axis_index("s")
    w = c * NS + s
    _fill(rows0, CH, FD, 0.0)
    for k in range(RPS // CH):
        pltpu.sync_copy(rows0, acc.at[pl.ds(s * RPS + k * CH, CH)])
    pltpu.sync_copy(srcs_hbm.at[w], src_v)
    pltpu.sync_copy(dsts_hbm.at[w], dst_v)
    plsc.subcore_barrier()

    def outer(r, carry):
        pltpu.async_copy(xs_hbm.at[src_v.at[r]], rows0, sg0).wait()
        pltpu.sync_copy(rows0, acc.at[dst_v.at[r]], add=True)
        return carry

    lax.fori_loop(0, IR, outer, 0)
    plsc.subcore_barrier()
    pltpu.sync_copy(acc.at[pl.ds(s * RPS, RPS)],
                    out_hbm.at[c].at[pl.ds(s * RPS, RPS)])


_BLK = 1000


def _scale_body(d0_ref, d1_ref, x_ref, xs_ref, dinvb_ref):
    deg = d0_ref[:, 0:1] + d1_ref[:, 0:1] + 1.0
    dinv = lax.rsqrt(deg)
    xs_ref[...] = x_ref[...] * dinv
    dinvb_ref[...] = jnp.broadcast_to(dinv, (_BLK, FD))


_scale_call = pl.pallas_call(
    _scale_body,
    grid=(NN // _BLK,),
    in_specs=[
        pl.BlockSpec((_BLK, FD), lambda i: (i, 0)),
        pl.BlockSpec((_BLK, FD), lambda i: (i, 0)),
        pl.BlockSpec((_BLK, FD), lambda i: (i, 0)),
    ],
    out_specs=[
        pl.BlockSpec((_BLK, FD), lambda i: (i, 0)),
        pl.BlockSpec((_BLK, FD), lambda i: (i, 0)),
    ],
    out_shape=[
        jax.ShapeDtypeStruct((NN, FD), jnp.float32),
        jax.ShapeDtypeStruct((NN, FD), jnp.float32),
    ],
)


def _dense_body(p0_ref, p1_ref, xs_ref, dinvb_ref, w1_ref, b1_ref, wmu_ref,
                h2s_ref):
    a1 = dinvb_ref[...] * (p0_ref[...] + p1_ref[...] + xs_ref[...])
    z = jnp.dot(a1, w1_ref[...], preferred_element_type=jnp.float32)
    z = z + b1_ref[...]
    z = jnp.where(z >= 0.0, z, 0.01 * z)
    h2 = jnp.dot(z, wmu_ref[...], preferred_element_type=jnp.float32)
    h2s_ref[...] = h2 * dinvb_ref[...]


_dense_call = pl.pallas_call(
    _dense_body,
    grid=(NN // _BLK,),
    in_specs=[
        pl.BlockSpec((_BLK, FD), lambda i: (i, 0)),
        pl.BlockSpec((_BLK, FD), lambda i: (i, 0)),
        pl.BlockSpec((_BLK, FD), lambda i: (i, 0)),
        pl.BlockSpec((_BLK, FD), lambda i: (i, 0)),
        pl.BlockSpec((FD, HD), lambda i: (0, 0)),
        pl.BlockSpec((1, HD), lambda i: (0, 0)),
        pl.BlockSpec((HD, FD), lambda i: (0, 0)),
    ],
    out_specs=pl.BlockSpec((_BLK, FD), lambda i: (i, 0)),
    out_shape=jax.ShapeDtypeStruct((NN, FD), jnp.float32),
)


def _out_body(q0_ref, q1_ref, h2s_ref, dinvb_ref, bmu_ref, mu_ref):
    mu_ref[...] = (
        dinvb_ref[...] * (q0_ref[...] + q1_ref[...] + h2s_ref[...])
        + bmu_ref[...]
    )


_out_call = pl.pallas_call(
    _out_body,
    grid=(NN // _BLK,),
    in_specs=[
        pl.BlockSpec((_BLK, FD), lambda i: (i, 0)),
        pl.BlockSpec((_BLK, FD), lambda i: (i, 0)),
        pl.BlockSpec((_BLK, FD), lambda i: (i, 0)),
        pl.BlockSpec((_BLK, FD), lambda i: (i, 0)),
        pl.BlockSpec((1, FD), lambda i: (0, 0)),
    ],
    out_specs=pl.BlockSpec((_BLK, FD), lambda i: (i, 0)),
    out_shape=jax.ShapeDtypeStruct((NN, FD), jnp.float32),
)


def kernel(x, edge_index, edge_type, W1, b1, W_mu, b_mu, W_logstd, b_logstd):
    pad = EPWP - EPW
    src = jnp.pad(edge_index[0].astype(jnp.int32).reshape(NW, EPW),
                  ((0, 0), (0, pad))).reshape(NW, IR, 128)
    dst = jnp.pad(edge_index[1].astype(jnp.int32).reshape(NW, EPW),
                  ((0, 0), (0, pad)),
                  constant_values=DUMP).reshape(NW, IR, 128)

    degp = _deg_kernel(dst)
    xs, dinvb = _scale_call(degp[0, :NN], degp[1, :NN], x)
    p = _agg_kernel(xs, src, dst)
    h2s = _dense_call(p[0, :NN], p[1, :NN], xs, dinvb,
                      W1, b1.reshape(1, HD), W_mu)
    q = _agg_kernel(h2s, src, dst)
    mu = _out_call(q[0, :NN], q[1, :NN], h2s, dinvb, b_mu.reshape(1, FD))
    return mu


# speedup vs baseline: 1.0026x; 1.0026x over previous
import functools

import jax
import jax.numpy as jnp
from jax import lax
from jax.experimental import pallas as pl
from jax.experimental.pallas import tpu as pltpu
from jax.experimental.pallas import tpu_sc as plsc

NN = 10000
EE = 320000
FD = 128
HD = 256
NC, NS = 2, 16
NW = NC * NS
EPW = EE // NW
EPWP = 10240
CH = 128
NCH = EPWP // CH
IR = EPWP // 128
ANN = 10240
RPS = ANN // NS
DUMP = 10200

_MESH = plsc.VectorSubcoreMesh(
    core_axis_name="c", subcore_axis_name="s", num_cores=NC, num_subcores=NS
)


def _fill(ref, nrows, ncols, value):
    v = jnp.full((16,), value, jnp.float32)

    def row(r, carry):
        for cc in range(ncols // 16):
            ref[r, pl.ds(cc * 16, 16)] = v
        return carry

    lax.fori_loop(0, nrows, row, 0)


@functools.partial(
    pl.kernel,
    out_type=jax.ShapeDtypeStruct((NC, ANN, FD), jnp.float32),
    mesh=_MESH,
    scratch_types=[
        pltpu.VMEM((IR, 128), jnp.int32),
        pltpu.VMEM((CH, FD), jnp.float32),
        pltpu.VMEM_SHARED((ANN, FD), jnp.float32),
    ],
)
def _deg_kernel(dsts_hbm, out_hbm, dst_v, ones_v, acc):
    c = lax.axis_index("c")
    s = lax.axis_index("s")
    w = c * NS + s
    _fill(ones_v, CH, FD, 0.0)
    for k in range(RPS // CH):
        pltpu.sync_copy(ones_v, acc.at[pl.ds(s * RPS + k * CH, CH)])
    _fill(ones_v, CH, FD, 1.0)
    pltpu.sync_copy(dsts_hbm.at[w], dst_v)
    plsc.subcore_barrier()

    def body(r, carry):
        pltpu.sync_copy(ones_v, acc.at[dst_v.at[r]], add=True)
        return carry

    lax.fori_loop(0, IR, body, 0)
    plsc.subcore_barrier()
    pltpu.sync_copy(acc.at[pl.ds(s * RPS, RPS)],
                    out_hbm.at[c].at[pl.ds(s * RPS, RPS)])


@functools.partial(
    pl.kernel,
    out_type=jax.ShapeDtypeStruct((NC, ANN, FD), jnp.float32),
    mesh=_MESH,
    scratch_types=[
        pltpu.VMEM((IR, 128), jnp.int32),
        pltpu.VMEM((IR, 128), jnp.int32),
        pltpu.VMEM((CH, FD), jnp.float32),
        pltpu.VMEM_SHARED((ANN, FD), jnp.float32),
        pltpu.SemaphoreType.DMA,
    ],
)
def _agg_kernel(xs_hbm, srcs_hbm, dsts_hbm, out_hbm,
                src_v, dst_v, rows0, acc, sg0):
    c = lax.axis_index("c")
    s = lax.axis_index("s")
    w = c * NS + s
    _fill(rows0, CH, FD, 0.0)
    for k in range(RPS // CH):
        pltpu.sync_copy(rows0, acc.at[pl.ds(s * RPS + k * CH, CH)])
    pltpu.sync_copy(srcs_hbm.at[w], src_v)
    pltpu.sync_copy(dsts_hbm.at[w], dst_v)
    plsc.subcore_barrier()

    def outer(r, carry):
        pltpu.async_copy(xs_hbm.at[src_v.at[r]], rows0, sg0).wait()
        pltpu.sync_copy(rows0, acc.at[dst_v.at[r]], add=True)
        return carry

    lax.fori_loop(0, IR, outer, 0)
    plsc.subcore_barrier()
    pltpu.sync_copy(acc.at[pl.ds(s * RPS, RPS)],
                    out_hbm.at[c].at[pl.ds(s * RPS, RPS)])


_BLK = 1000


def _scale_body(d0_ref, d1_ref, x_ref, xs_ref, dinvb_ref):
    deg = d0_ref[:, 0:1] + d1_ref[:, 0:1] + 1.0
    dinv = lax.rsqrt(deg)
    xs_ref[...] = x_ref[...] * dinv
    dinvb_ref[...] = jnp.broadcast_to(dinv, (_BLK, FD))


_scale_call = pl.pallas_call(
    _scale_body,
    grid=(NN // _BLK,),
    in_specs=[
        pl.BlockSpec((_BLK, FD), lambda i: (i, 0)),
        pl.BlockSpec((_BLK, FD), lambda i: (i, 0)),
        pl.BlockSpec((_BLK, FD), lambda i: (i, 0)),
    ],
    out_specs=[
        pl.BlockSpec((_BLK, FD), lambda i: (i, 0)),
        pl.BlockSpec((_BLK, FD), lambda i: (i, 0)),
    ],
    out_shape=[
        jax.ShapeDtypeStruct((NN, FD), jnp.float32),
        jax.ShapeDtypeStruct((NN, FD), jnp.float32),
    ],
)


def _dense_body(p0_ref, p1_ref, xs_ref, dinvb_ref, w1_ref, b1_ref, wmu_ref,
                h2s_ref):
    a1 = dinvb_ref[...] * (p0_ref[...] + p1_ref[...] + xs_ref[...])
    z = jnp.dot(a1, w1_ref[...], preferred_element_type=jnp.float32)
    z = z + b1_ref[...]
    z = jnp.where(z >= 0.0, z, 0.01 * z)
    h2 = jnp.dot(z, wmu_ref[...], preferred_element_type=jnp.float32)
    h2s_ref[...] = h2 * dinvb_ref[...]


_dense_call = pl.pallas_call(
    _dense_body,
    grid=(NN // _BLK,),
    in_specs=[
        pl.BlockSpec((_BLK, FD), lambda i: (i, 0)),
        pl.BlockSpec((_BLK, FD), lambda i: (i, 0)),
        pl.BlockSpec((_BLK, FD), lambda i: (i, 0)),
        pl.BlockSpec((_BLK, FD), lambda i: (i, 0)),
        pl.BlockSpec((FD, HD), lambda i: (0, 0)),
        pl.BlockSpec((1, HD), lambda i: (0, 0)),
        pl.BlockSpec((HD, FD), lambda i: (0, 0)),
    ],
    out_specs=pl.BlockSpec((_BLK, FD), lambda i: (i, 0)),
    out_shape=jax.ShapeDtypeStruct((NN, FD), jnp.float32),
)


def _out_body(q0_ref, q1_ref, h2s_ref, dinvb_ref, bmu_ref, mu_ref):
    mu_ref[...] = (
        dinvb_ref[...] * (q0_ref[...] + q1_ref[...] + h2s_ref[...])
        + bmu_ref[...]
    )


_out_call = pl.pallas_call(
    _out_body,
    grid=(NN // _BLK,),
    in_specs=[
        pl.BlockSpec((_BLK, FD), lambda i: (i, 0)),
        pl.BlockSpec((_BLK, FD), lambda i: (i, 0)),
        pl.BlockSpec((_BLK, FD), lambda i: (i, 0)),
        pl.BlockSpec((_BLK, FD), lambda i: (i, 0)),
        pl.BlockSpec((1, FD), lambda i: (0, 0)),
    ],
    out_specs=pl.BlockSpec((_BLK, FD), lambda i: (i, 0)),
    out_shape=jax.ShapeDtypeStruct((NN, FD), jnp.float32),
)


def kernel(x, edge_index, edge_type, W1, b1, W_mu, b_mu, W_logstd, b_logstd):
    pad = EPWP - EPW
    src = jnp.pad(edge_index[0].astype(jnp.int32).reshape(NW, EPW),
                  ((0, 0), (0, pad))).reshape(NW, IR, 128)
    dump = jnp.broadcast_to(NN + jnp.arange(pad, dtype=jnp.int32), (NW, pad))
    dst = jnp.concatenate(
        [edge_index[1].astype(jnp.int32).reshape(NW, EPW), dump],
        axis=1).reshape(NW, IR, 128)

    degp = _deg_kernel(dst)
    xs, dinvb = _scale_call(degp[0, :NN], degp[1, :NN], x)
    p = _agg_kernel(xs, src, dst)
    h2s = _dense_call(p[0, :NN], p[1, :NN], xs, dinvb,
                      W1, b1.reshape(1, HD), W_mu)
    q = _agg_kernel(h2s, src, dst)
    mu = _out_call(q[0, :NN], q[1, :NN], h2s, dinvb, b_mu.reshape(1, FD))
    return mu


# speedup vs baseline: 1.8872x; 1.8824x over previous
import functools

import jax
import jax.numpy as jnp
from jax import lax
from jax.experimental import pallas as pl
from jax.experimental.pallas import tpu as pltpu
from jax.experimental.pallas import tpu_sc as plsc

NN = 10000
EE = 320000
FD = 128
HD = 256
NC, NS = 2, 16
NW = NC * NS
EPW = EE // NW
CH = 80
NCH = EPW // CH
ANN = 10240
RPS = ANN // NS

_MESH = plsc.VectorSubcoreMesh(
    core_axis_name="c", subcore_axis_name="s", num_cores=NC, num_subcores=NS
)


def _fill(ref, nrows, ncols, value):
    v = jnp.full((16,), value, jnp.float32)

    def row(r, carry):
        for cc in range(ncols // 16):
            ref[r, pl.ds(cc * 16, 16)] = v
        return carry

    lax.fori_loop(0, nrows, row, 0)


@functools.partial(
    pl.kernel,
    out_type=jax.ShapeDtypeStruct((NC, ANN, FD), jnp.float32),
    mesh=_MESH,
    scratch_types=[
        pltpu.VMEM((NCH, CH), jnp.int32),
        pltpu.VMEM((CH, FD), jnp.float32),
        pltpu.VMEM_SHARED((ANN, FD), jnp.float32),
    ],
)
def _deg_kernel(dsts_hbm, out_hbm, dst_v, ones_v, acc):
    c = lax.axis_index("c")
    s = lax.axis_index("s")
    w = c * NS + s
    _fill(ones_v, CH, FD, 0.0)
    for k in range(RPS // CH):
        pltpu.sync_copy(ones_v, acc.at[pl.ds(s * RPS + k * CH, CH)])
    _fill(ones_v, CH, FD, 1.0)
    pltpu.sync_copy(dsts_hbm.at[w], dst_v)
    plsc.subcore_barrier()

    def body(j, carry):
        pltpu.sync_copy(ones_v, acc.at[dst_v.at[j]], add=True)
        return carry

    lax.fori_loop(0, NCH, body, 0)
    plsc.subcore_barrier()
    pltpu.sync_copy(acc.at[pl.ds(s * RPS, RPS)],
                    out_hbm.at[c].at[pl.ds(s * RPS, RPS)])


@functools.partial(
    pl.kernel,
    out_type=jax.ShapeDtypeStruct((NC, ANN, FD), jnp.float32),
    mesh=_MESH,
    scratch_types=[
        pltpu.VMEM((NCH, CH), jnp.int32),
        pltpu.VMEM((NCH, CH), jnp.int32),
        pltpu.VMEM((CH, FD), jnp.float32),
        pltpu.VMEM_SHARED((ANN, FD), jnp.float32),
        pltpu.SemaphoreType.DMA,
    ],
)
def _agg_kernel(xs_hbm, srcs_hbm, dsts_hbm, out_hbm,
                src_v, dst_v, rows0, acc, sg0):
    c = lax.axis_index("c")
    s = lax.axis_index("s")
    w = c * NS + s
    _fill(rows0, CH, FD, 0.0)
    for k in range(RPS // CH):
        pltpu.sync_copy(rows0, acc.at[pl.ds(s * RPS + k * CH, CH)])
    pltpu.sync_copy(srcs_hbm.at[w], src_v)
    pltpu.sync_copy(dsts_hbm.at[w], dst_v)
    plsc.subcore_barrier()

    def outer(j, carry):
        pltpu.async_copy(xs_hbm.at[src_v.at[j]], rows0, sg0).wait()
        pltpu.sync_copy(rows0, acc.at[dst_v.at[j]], add=True)
        return carry

    lax.fori_loop(0, NCH, outer, 0)
    plsc.subcore_barrier()
    pltpu.sync_copy(acc.at[pl.ds(s * RPS, RPS)],
                    out_hbm.at[c].at[pl.ds(s * RPS, RPS)])


_BLK = 1000


def _scale_body(d0_ref, d1_ref, x_ref, xs_ref, dinvb_ref):
    deg = d0_ref[:, 0:1] + d1_ref[:, 0:1] + 1.0
    dinv = lax.rsqrt(deg)
    xs_ref[...] = x_ref[...] * dinv
    dinvb_ref[...] = jnp.broadcast_to(dinv, (_BLK, FD))


_scale_call = pl.pallas_call(
    _scale_body,
    grid=(NN // _BLK,),
    in_specs=[
        pl.BlockSpec((_BLK, FD), lambda i: (i, 0)),
        pl.BlockSpec((_BLK, FD), lambda i: (i, 0)),
        pl.BlockSpec((_BLK, FD), lambda i: (i, 0)),
    ],
    out_specs=[
        pl.BlockSpec((_BLK, FD), lambda i: (i, 0)),
        pl.BlockSpec((_BLK, FD), lambda i: (i, 0)),
    ],
    out_shape=[
        jax.ShapeDtypeStruct((NN, FD), jnp.float32),
        jax.ShapeDtypeStruct((NN, FD), jnp.float32),
    ],
)


def _dense_body(p0_ref, p1_ref, xs_ref, dinvb_ref, w1_ref, b1_ref, wmu_ref,
                h2s_ref):
    a1 = dinvb_ref[...] * (p0_ref[...] + p1_ref[...] + xs_ref[...])
    z = jnp.dot(a1, w1_ref[...], preferred_element_type=jnp.float32)
    z = z + b1_ref[...]
    z = jnp.where(z >= 0.0, z, 0.01 * z)
    h2 = jnp.dot(z, wmu_ref[...], preferred_element_type=jnp.float32)
    h2s_ref[...] = h2 * dinvb_ref[...]


_dense_call = pl.pallas_call(
    _dense_body,
    grid=(NN // _BLK,),
    in_specs=[
        pl.BlockSpec((_BLK, FD), lambda i: (i, 0)),
        pl.BlockSpec((_BLK, FD), lambda i: (i, 0)),
        pl.BlockSpec((_BLK, FD), lambda i: (i, 0)),
        pl.BlockSpec((_BLK, FD), lambda i: (i, 0)),
        pl.BlockSpec((FD, HD), lambda i: (0, 0)),
        pl.BlockSpec((1, HD), lambda i: (0, 0)),
        pl.BlockSpec((HD, FD), lambda i: (0, 0)),
    ],
    out_specs=pl.BlockSpec((_BLK, FD), lambda i: (i, 0)),
    out_shape=jax.ShapeDtypeStruct((NN, FD), jnp.float32),
)


def _out_body(q0_ref, q1_ref, h2s_ref, dinvb_ref, bmu_ref, mu_ref):
    mu_ref[...] = (
        dinvb_ref[...] * (q0_ref[...] + q1_ref[...] + h2s_ref[...])
        + bmu_ref[...]
    )


_out_call = pl.pallas_call(
    _out_body,
    grid=(NN // _BLK,),
    in_specs=[
        pl.BlockSpec((_BLK, FD), lambda i: (i, 0)),
        pl.BlockSpec((_BLK, FD), lambda i: (i, 0)),
        pl.BlockSpec((_BLK, FD), lambda i: (i, 0)),
        pl.BlockSpec((_BLK, FD), lambda i: (i, 0)),
        pl.BlockSpec((1, FD), lambda i: (0, 0)),
    ],
    out_specs=pl.BlockSpec((_BLK, FD), lambda i: (i, 0)),
    out_shape=jax.ShapeDtypeStruct((NN, FD), jnp.float32),
)


def kernel(x, edge_index, edge_type, W1, b1, W_mu, b_mu, W_logstd, b_logstd):
    src = edge_index[0].astype(jnp.int32).reshape(NW, NCH, CH)
    dst = edge_index[1].astype(jnp.int32).reshape(NW, NCH, CH)

    degp = _deg_kernel(dst)
    xs, dinvb = _scale_call(degp[0, :NN], degp[1, :NN], x)
    p = _agg_kernel(xs, src, dst)
    h2s = _dense_call(p[0, :NN], p[1, :NN], xs, dinvb,
                      W1, b1.reshape(1, HD), W_mu)
    q = _agg_kernel(h2s, src, dst)
    mu = _out_call(q[0, :NN], q[1, :NN], h2s, dinvb, b_mu.reshape(1, FD))
    return mu


# speedup vs baseline: 2.0155x; 1.0680x over previous
import functools

import jax
import jax.numpy as jnp
from jax import lax
from jax.experimental import pallas as pl
from jax.experimental.pallas import tpu as pltpu
from jax.experimental.pallas import tpu_sc as plsc

NN = 10000
EE = 320000
FD = 128
HD = 256
NC, NS = 2, 16
NW = NC * NS
EPW = EE // NW
CH = 100
NCH = EPW // CH
ANN = 10240
RPS = ANN // NS

_MESH = plsc.VectorSubcoreMesh(
    core_axis_name="c", subcore_axis_name="s", num_cores=NC, num_subcores=NS
)


def _fill(ref, nrows, ncols, value):
    v = jnp.full((16,), value, jnp.float32)

    def row(r, carry):
        for cc in range(ncols // 16):
            ref[r, pl.ds(cc * 16, 16)] = v
        return carry

    lax.fori_loop(0, nrows, row, 0)


@functools.partial(
    pl.kernel,
    out_type=jax.ShapeDtypeStruct((NC, ANN, FD), jnp.float32),
    mesh=_MESH,
    scratch_types=[
        pltpu.VMEM((NCH, CH), jnp.int32),
        pltpu.VMEM((CH, FD), jnp.float32),
        pltpu.VMEM_SHARED((ANN, FD), jnp.float32),
    ],
)
def _deg_kernel(dsts_hbm, out_hbm, dst_v, ones_v, acc):
    c = lax.axis_index("c")
    s = lax.axis_index("s")
    w = c * NS + s
    _fill(ones_v, CH, FD, 0.0)
    for k in range(RPS // CH):
        pltpu.sync_copy(ones_v, acc.at[pl.ds(s * RPS + k * CH, CH)])
    pltpu.sync_copy(ones_v.at[pl.ds(0, RPS % CH)],
                    acc.at[pl.ds(s * RPS + RPS - RPS % CH, RPS % CH)])
    _fill(ones_v, CH, FD, 1.0)
    pltpu.sync_copy(dsts_hbm.at[w], dst_v)
    plsc.subcore_barrier()

    def body(j, carry):
        pltpu.sync_copy(ones_v, acc.at[dst_v.at[j]], add=True)
        return carry

    lax.fori_loop(0, NCH, body, 0)
    plsc.subcore_barrier()
    pltpu.sync_copy(acc.at[pl.ds(s * RPS, RPS)],
                    out_hbm.at[c].at[pl.ds(s * RPS, RPS)])


@functools.partial(
    pl.kernel,
    out_type=jax.ShapeDtypeStruct((NC, ANN, FD), jnp.float32),
    mesh=_MESH,
    scratch_types=[
        pltpu.VMEM((NCH, CH), jnp.int32),
        pltpu.VMEM((NCH, CH), jnp.int32),
        pltpu.VMEM((CH, FD), jnp.float32),
        pltpu.VMEM_SHARED((ANN, FD), jnp.float32),
        pltpu.SemaphoreType.DMA,
    ],
)
def _agg_kernel(xs_hbm, srcs_hbm, dsts_hbm, out_hbm,
                src_v, dst_v, rows0, acc, sg0):
    c = lax.axis_index("c")
    s = lax.axis_index("s")
    w = c * NS + s
    _fill(rows0, CH, FD, 0.0)
    for k in range(RPS // CH):
        pltpu.sync_copy(rows0, acc.at[pl.ds(s * RPS + k * CH, CH)])
    pltpu.sync_copy(rows0.at[pl.ds(0, RPS % CH)],
                    acc.at[pl.ds(s * RPS + RPS - RPS % CH, RPS % CH)])
    pltpu.sync_copy(srcs_hbm.at[w], src_v)
    pltpu.sync_copy(dsts_hbm.at[w], dst_v)
    plsc.subcore_barrier()

    def outer(j, carry):
        pltpu.async_copy(xs_hbm.at[src_v.at[j]], rows0, sg0).wait()
        pltpu.sync_copy(rows0, acc.at[dst_v.at[j]], add=True)
        return carry

    lax.fori_loop(0, NCH, outer, 0)
    plsc.subcore_barrier()
    pltpu.sync_copy(acc.at[pl.ds(s * RPS, RPS)],
                    out_hbm.at[c].at[pl.ds(s * RPS, RPS)])


_BLK = 1000


def _scale_body(d0_ref, d1_ref, x_ref, xs_ref, dinvb_ref):
    deg = d0_ref[:, 0:1] + d1_ref[:, 0:1] + 1.0
    dinv = lax.rsqrt(deg)
    xs_ref[...] = x_ref[...] * dinv
    dinvb_ref[...] = jnp.broadcast_to(dinv, (_BLK, FD))


_scale_call = pl.pallas_call(
    _scale_body,
    grid=(NN // _BLK,),
    in_specs=[
        pl.BlockSpec((_BLK, FD), lambda i: (i, 0)),
        pl.BlockSpec((_BLK, FD), lambda i: (i, 0)),
        pl.BlockSpec((_BLK, FD), lambda i: (i, 0)),
    ],
    out_specs=[
        pl.BlockSpec((_BLK, FD), lambda i: (i, 0)),
        pl.BlockSpec((_BLK, FD), lambda i: (i, 0)),
    ],
    out_shape=[
        jax.ShapeDtypeStruct((NN, FD), jnp.float32),
        jax.ShapeDtypeStruct((NN, FD), jnp.float32),
    ],
)


def _dense_body(p0_ref, p1_ref, xs_ref, dinvb_ref, w1_ref, b1_ref, wmu_ref,
                h2s_ref):
    a1 = dinvb_ref[...] * (p0_ref[...] + p1_ref[...] + xs_ref[...])
    z = jnp.dot(a1, w1_ref[...], preferred_element_type=jnp.float32)
    z = z + b1_ref[...]
    z = jnp.where(z >= 0.0, z, 0.01 * z)
    h2 = jnp.dot(z, wmu_ref[...], preferred_element_type=jnp.float32)
    h2s_ref[...] = h2 * dinvb_ref[...]


_dense_call = pl.pallas_call(
    _dense_body,
    grid=(NN // _BLK,),
    in_specs=[
        pl.BlockSpec((_BLK, FD), lambda i: (i, 0)),
        pl.BlockSpec((_BLK, FD), lambda i: (i, 0)),
        pl.BlockSpec((_BLK, FD), lambda i: (i, 0)),
        pl.BlockSpec((_BLK, FD), lambda i: (i, 0)),
        pl.BlockSpec((FD, HD), lambda i: (0, 0)),
        pl.BlockSpec((1, HD), lambda i: (0, 0)),
        pl.BlockSpec((HD, FD), lambda i: (0, 0)),
    ],
    out_specs=pl.BlockSpec((_BLK, FD), lambda i: (i, 0)),
    out_shape=jax.ShapeDtypeStruct((NN, FD), jnp.float32),
)


def _out_body(q0_ref, q1_ref, h2s_ref, dinvb_ref, bmu_ref, mu_ref):
    mu_ref[...] = (
        dinvb_ref[...] * (q0_ref[...] + q1_ref[...] + h2s_ref[...])
        + bmu_ref[...]
    )


_out_call = pl.pallas_call(
    _out_body,
    grid=(NN // _BLK,),
    in_specs=[
        pl.BlockSpec((_BLK, FD), lambda i: (i, 0)),
        pl.BlockSpec((_BLK, FD), lambda i: (i, 0)),
        pl.BlockSpec((_BLK, FD), lambda i: (i, 0)),
        pl.BlockSpec((_BLK, FD), lambda i: (i, 0)),
        pl.BlockSpec((1, FD), lambda i: (0, 0)),
    ],
    out_specs=pl.BlockSpec((_BLK, FD), lambda i: (i, 0)),
    out_shape=jax.ShapeDtypeStruct((NN, FD), jnp.float32),
)


def kernel(x, edge_index, edge_type, W1, b1, W_mu, b_mu, W_logstd, b_logstd):
    src = edge_index[0].astype(jnp.int32).reshape(NW, NCH, CH)
    dst = edge_index[1].astype(jnp.int32).reshape(NW, NCH, CH)

    degp = _deg_kernel(dst)
    xs, dinvb = _scale_call(degp[0, :NN], degp[1, :NN], x)
    p = _agg_kernel(xs, src, dst)
    h2s = _dense_call(p[0, :NN], p[1, :NN], xs, dinvb,
                      W1, b1.reshape(1, HD), W_mu)
    q = _agg_kernel(h2s, src, dst)
    mu = _out_call(q[0, :NN], q[1, :NN], h2s, dinvb, b_mu.reshape(1, FD))
    return mu


# speedup vs baseline: 2.1213x; 1.0525x over previous
import functools

import jax
import jax.numpy as jnp
from jax import lax
from jax.experimental import pallas as pl
from jax.experimental.pallas import tpu as pltpu
from jax.experimental.pallas import tpu_sc as plsc

NN = 10000
EE = 320000
FD = 128
HD = 256
NC, NS = 2, 16
NW = NC * NS
EPW = EE // NW
CH = 125
NCH = EPW // CH
ANN = 10240
RPS = ANN // NS

_MESH = plsc.VectorSubcoreMesh(
    core_axis_name="c", subcore_axis_name="s", num_cores=NC, num_subcores=NS
)


def _fill(ref, nrows, ncols, value):
    v = jnp.full((16,), value, jnp.float32)

    def row(r, carry):
        for cc in range(ncols // 16):
            ref[r, pl.ds(cc * 16, 16)] = v
        return carry

    lax.fori_loop(0, nrows, row, 0)


@functools.partial(
    pl.kernel,
    out_type=jax.ShapeDtypeStruct((NC, ANN, FD), jnp.float32),
    mesh=_MESH,
    scratch_types=[
        pltpu.VMEM((NCH, CH), jnp.int32),
        pltpu.VMEM((CH, FD), jnp.float32),
        pltpu.VMEM_SHARED((ANN, FD), jnp.float32),
    ],
)
def _deg_kernel(dsts_hbm, out_hbm, dst_v, ones_v, acc):
    c = lax.axis_index("c")
    s = lax.axis_index("s")
    w = c * NS + s
    _fill(ones_v, CH, FD, 0.0)
    for k in range(5):
        pltpu.sync_copy(ones_v.at[pl.ds(0, 120)],
                        acc.at[pl.ds(s * RPS + k * 120, 120)])
    pltpu.sync_copy(ones_v.at[pl.ds(0, 40)],
                    acc.at[pl.ds(s * RPS + 600, 40)])
    _fill(ones_v, CH, FD, 1.0)
    pltpu.sync_copy(dsts_hbm.at[w], dst_v)
    plsc.subcore_barrier()

    def body(j, carry):
        pltpu.sync_copy(ones_v, acc.at[dst_v.at[j]], add=True)
        return carry

    lax.fori_loop(0, NCH, body, 0)
    plsc.subcore_barrier()
    pltpu.sync_copy(acc.at[pl.ds(s * RPS, RPS)],
                    out_hbm.at[c].at[pl.ds(s * RPS, RPS)])


@functools.partial(
    pl.kernel,
    out_type=jax.ShapeDtypeStruct((NC, ANN, FD), jnp.float32),
    mesh=_MESH,
    scratch_types=[
        pltpu.VMEM((NCH, CH), jnp.int32),
        pltpu.VMEM((NCH, CH), jnp.int32),
        pltpu.VMEM((CH, FD), jnp.float32),
        pltpu.VMEM_SHARED((ANN, FD), jnp.float32),
        pltpu.SemaphoreType.DMA,
    ],
)
def _agg_kernel(xs_hbm, srcs_hbm, dsts_hbm, out_hbm,
                src_v, dst_v, rows0, acc, sg0):
    c = lax.axis_index("c")
    s = lax.axis_index("s")
    w = c * NS + s
    _fill(rows0, CH, FD, 0.0)
    for k in range(5):
        pltpu.sync_copy(rows0.at[pl.ds(0, 120)],
                        acc.at[pl.ds(s * RPS + k * 120, 120)])
    pltpu.sync_copy(rows0.at[pl.ds(0, 40)],
                    acc.at[pl.ds(s * RPS + 600, 40)])
    pltpu.sync_copy(srcs_hbm.at[w], src_v)
    pltpu.sync_copy(dsts_hbm.at[w], dst_v)
    plsc.subcore_barrier()

    def outer(j, carry):
        pltpu.async_copy(xs_hbm.at[src_v.at[j]], rows0, sg0).wait()
        pltpu.sync_copy(rows0, acc.at[dst_v.at[j]], add=True)
        return carry

    lax.fori_loop(0, NCH, outer, 0)
    plsc.subcore_barrier()
    pltpu.sync_copy(acc.at[pl.ds(s * RPS, RPS)],
                    out_hbm.at[c].at[pl.ds(s * RPS, RPS)])


_BLK = 1000


def _scale_body(d0_ref, d1_ref, x_ref, xs_ref, dinvb_ref):
    deg = d0_ref[:, 0:1] + d1_ref[:, 0:1] + 1.0
    dinv = lax.rsqrt(deg)
    xs_ref[...] = x_ref[...] * dinv
    dinvb_ref[...] = jnp.broadcast_to(dinv, (_BLK, FD))


_scale_call = pl.pallas_call(
    _scale_body,
    grid=(NN // _BLK,),
    in_specs=[
        pl.BlockSpec((_BLK, FD), lambda i: (i, 0)),
        pl.BlockSpec((_BLK, FD), lambda i: (i, 0)),
        pl.BlockSpec((_BLK, FD), lambda i: (i, 0)),
    ],
    out_specs=[
        pl.BlockSpec((_BLK, FD), lambda i: (i, 0)),
        pl.BlockSpec((_BLK, FD), lambda i: (i, 0)),
    ],
    out_shape=[
        jax.ShapeDtypeStruct((NN, FD), jnp.float32),
        jax.ShapeDtypeStruct((NN, FD), jnp.float32),
    ],
)


def _dense_body(p0_ref, p1_ref, xs_ref, dinvb_ref, w1_ref, b1_ref, wmu_ref,
                h2s_ref):
    a1 = dinvb_ref[...] * (p0_ref[...] + p1_ref[...] + xs_ref[...])
    z = jnp.dot(a1, w1_ref[...], preferred_element_type=jnp.float32)
    z = z + b1_ref[...]
    z = jnp.where(z >= 0.0, z, 0.01 * z)
    h2 = jnp.dot(z, wmu_ref[...], preferred_element_type=jnp.float32)
    h2s_ref[...] = h2 * dinvb_ref[...]


_dense_call = pl.pallas_call(
    _dense_body,
    grid=(NN // _BLK,),
    in_specs=[
        pl.BlockSpec((_BLK, FD), lambda i: (i, 0)),
        pl.BlockSpec((_BLK, FD), lambda i: (i, 0)),
        pl.BlockSpec((_BLK, FD), lambda i: (i, 0)),
        pl.BlockSpec((_BLK, FD), lambda i: (i, 0)),
        pl.BlockSpec((FD, HD), lambda i: (0, 0)),
        pl.BlockSpec((1, HD), lambda i: (0, 0)),
        pl.BlockSpec((HD, FD), lambda i: (0, 0)),
    ],
    out_specs=pl.BlockSpec((_BLK, FD), lambda i: (i, 0)),
    out_shape=jax.ShapeDtypeStruct((NN, FD), jnp.float32),
)


def _out_body(q0_ref, q1_ref, h2s_ref, dinvb_ref, bmu_ref, mu_ref):
    mu_ref[...] = (
        dinvb_ref[...] * (q0_ref[...] + q1_ref[...] + h2s_ref[...])
        + bmu_ref[...]
    )


_out_call = pl.pallas_call(
    _out_body,
    grid=(NN // _BLK,),
    in_specs=[
        pl.BlockSpec((_BLK, FD), lambda i: (i, 0)),
        pl.BlockSpec((_BLK, FD), lambda i: (i, 0)),
        pl.BlockSpec((_BLK, FD), lambda i: (i, 0)),
        pl.BlockSpec((_BLK, FD), lambda i: (i, 0)),
        pl.BlockSpec((1, FD), lambda i: (0, 0)),
    ],
    out_specs=pl.BlockSpec((_BLK, FD), lambda i: (i, 0)),
    out_shape=jax.ShapeDtypeStruct((NN, FD), jnp.float32),
)


def kernel(x, edge_index, edge_type, W1, b1, W_mu, b_mu, W_logstd, b_logstd):
    src = edge_index[0].astype(jnp.int32).reshape(NW, NCH, CH)
    dst = edge_index[1].astype(jnp.int32).reshape(NW, NCH, CH)

    degp = _deg_kernel(dst)
    xs, dinvb = _scale_call(degp[0, :NN], degp[1, :NN], x)
    p = _agg_kernel(xs, src, dst)
    h2s = _dense_call(p[0, :NN], p[1, :NN], xs, dinvb,
                      W1, b1.reshape(1, HD), W_mu)
    q = _agg_kernel(h2s, src, dst)
    mu = _out_call(q[0, :NN], q[1, :NN], h2s, dinvb, b_mu.reshape(1, FD))
    return mu
